# baseline (speedup 1.0000x reference)
import functools
import math

import jax
import jax.numpy as jnp
from jax import lax
from jax.experimental import pallas as pl
from jax.experimental.pallas import tpu as pltpu
from jax.experimental.pallas import tpu_sc as plsc

_B = 16384
_NF = 26
_VOCAB = 100000
_ED = 50
_EDP = 64
_EPS = 1e-5
_INV = 1.0 / math.sqrt(1.0 + _EPS)

_NC, _NS = 2, 16
_NW = _NC * _NS

_VT = _VOCAB // 128
_VREM = _VOCAB - _VT * 128
_NQ = _NF * _VT
_QPW = (_NQ + _NW - 1) // _NW


def _iota16():
    return lax.iota(jnp.int32, 16)


def _t16(xs):
    lane = _iota16()
    for s in (8, 4, 2, 1):
        m = (lane & s) == 0
        up_ix = (lane - s) & 15
        dn_ix = (lane + s) & 15
        ys = list(xs)
        for r in range(16):
            if r & s == 0:
                p, q = xs[r], xs[r + s]
                ys[r] = jnp.where(
                    m, p, q.at[up_ix].get(mode="promise_in_bounds"))
                ys[r + s] = jnp.where(
                    m, p.at[dn_ix].get(mode="promise_in_bounds"), q)
        xs = ys
    return xs


def _tp_block(buf, obuf, nv):
    def vblock(vb2, carry):
        for u in range(2):
            vb = vb2 * 2 + u
            v0 = vb * 16
            for eb in range(4):
                e0 = eb * 16
                xs = [buf[e0 + r, pl.ds(v0, 16)] for r in range(16)]
                ys = _t16(xs)
                for i in range(16):
                    obuf[vb * 8 + i // 2,
                         pl.ds((i % 2) * 64 + e0, 16)] = ys[i]
        return carry

    lax.fori_loop(0, nv // 32, vblock, 0)


_QFULL = _NQ // _NW


def _sc_transpose_body(tt_hbm, tail_hbm, out_hbm,
                       buf_a, buf_b, obuf_a, obuf_b, gsem, ssem):
    wid = lax.axis_index("s") * _NC + lax.axis_index("c")

    def start_in(q, buf):
        f = q // _VT
        c = q % _VT
        d1 = pltpu.async_copy(tt_hbm.at[f, pl.ds(0, 48), pl.ds(c * 128, 128)],
                              buf.at[pl.ds(0, 48)], gsem)
        d2 = pltpu.async_copy(tt_hbm.at[f, pl.ds(48, 2), pl.ds(c * 128, 128)],
                              buf.at[pl.ds(48, 2)], gsem)
        return d1, d2

    def start_out(q, obuf):
        f = q // _VT
        c = q % _VT
        return pltpu.async_copy(
            obuf, out_hbm.at[pl.ds(f * (_VOCAB // 2) + c * 64, 64)], ssem)

    def step(k2, carry):
        qa = wid + (2 * k2) * _NW
        qb = wid + (2 * k2 + 1) * _NW
        da = start_in(qa, buf_a)
        db = start_in(qb, buf_b)
        da[0].wait()
        da[1].wait()
        _tp_block(buf_a, obuf_a, 128)
        sa = start_out(qa, obuf_a)
        db[0].wait()
        db[1].wait()
        _tp_block(buf_b, obuf_b, 128)
        sb = start_out(qb, obuf_b)
        sa.wait()
        sb.wait()
        return carry

    lax.fori_loop(0, _QFULL // 2, step, 0)

    q_last = wid + _QFULL * _NW

    @pl.when(q_last < _NQ)
    def _():
        da = start_in(q_last, buf_a)
        da[0].wait()
        da[1].wait()
        _tp_block(buf_a, obuf_a, 128)
        start_out(q_last, obuf_a).wait()

    @pl.when(wid < _NF)
    def _():
        f = wid
        pltpu.sync_copy(tail_hbm.at[f], obuf_b.at[pl.ds(0, 16)])
        pltpu.sync_copy(
            obuf_b.at[pl.ds(0, 16)],
            out_hbm.at[pl.ds(f * (_VOCAB // 2) + _VT * 64, 16)])


@functools.cache
def _sc_transpose():
    mesh = plsc.VectorSubcoreMesh(
        core_axis_name="c", subcore_axis_name="s",
        num_cores=_NC, num_subcores=_NS)
    return pl.kernel(
        _sc_transpose_body,
        out_type=jax.ShapeDtypeStruct((_NF * _VOCAB // 2, 128), jnp.float32),
        mesh=mesh,
        scratch_types=[
            pltpu.VMEM((64, 128), jnp.float32),
            pltpu.VMEM((64, 128), jnp.float32),
            pltpu.VMEM((64, 128), jnp.float32),
            pltpu.VMEM((64, 128), jnp.float32),
            pltpu.SemaphoreType.DMA,
            pltpu.SemaphoreType.DMA,
        ],
        compiler_params=pltpu.CompilerParams(
            use_tc_tiling_on_sc=True, needs_layout_passes=False),
    )


_ROWS = _B * _NF
_RPW = _ROWS // _NW
_CHUNK = 128
_NCH = _RPW // _CHUNK
_K = 8


def _sc_gather_body(table_hbm, idx_hbm, out_hbm, idx_v, rows_v, gsem, ssem):
    wid = lax.axis_index("s") * _NC + lax.axis_index("c")
    pltpu.sync_copy(idx_hbm.at[pl.ds(wid * _NCH, _NCH)], idx_v)

    def group(g, carry):
        c0 = g * _K
        gd = []
        for b in range(_K):
            gd.append(pltpu.async_copy(
                table_hbm.at[idx_v.at[c0 + b]], rows_v.at[b], gsem))
        sd = []
        for b in range(_K):
            gd[b].wait()
            sd.append(pltpu.async_copy(
                rows_v.at[b],
                out_hbm.at[pl.ds((wid * _NCH + c0 + b) * _CHUNK, _CHUNK)],
                ssem))
        for b in range(_K):
            sd[b].wait()
        return carry

    lax.fori_loop(0, _NCH // _K, group, 0)


@functools.cache
def _sc_gather():
    mesh = plsc.VectorSubcoreMesh(
        core_axis_name="c", subcore_axis_name="s",
        num_cores=_NC, num_subcores=_NS)
    return pl.kernel(
        _sc_gather_body,
        out_type=jax.ShapeDtypeStruct((_ROWS, _EDP), jnp.float32),
        mesh=mesh,
        scratch_types=[
            pltpu.VMEM((_NCH, _CHUNK), jnp.int32),
            pltpu.VMEM((_K, _CHUNK, _EDP), jnp.float32),
            pltpu.SemaphoreType.DMA,
            pltpu.SemaphoreType.DMA,
        ],
        compiler_params=pltpu.CompilerParams(use_tc_tiling_on_sc=False),
    )


_BLK = 512
_E1 = _NF * _EDP
_ND = 13
_H1, _H2, _H3 = 512, 256, 128


def _mlp_body(emb_ref, num_ref, w1e_ref, w1n_ref, b1_ref, g1_ref, be1_ref,
              w2_ref, b2_ref, g2_ref, be2_ref,
              w3_ref, b3_ref, g3_ref, be3_ref,
              w4_ref, b4_ref, out_ref):
    h = jnp.dot(emb_ref[...], w1e_ref[...], preferred_element_type=jnp.float32)
    h = h + jnp.dot(num_ref[...], w1n_ref[...],
                    preferred_element_type=jnp.float32)
    h = ((h + b1_ref[...]) * _INV) * g1_ref[...] + be1_ref[...]
    x = jnp.maximum(h, 0.0)

    h = jnp.dot(x, w2_ref[...], preferred_element_type=jnp.float32)
    h = ((h + b2_ref[...]) * _INV) * g2_ref[...] + be2_ref[...]
    x = jnp.maximum(h, 0.0)

    h = jnp.dot(x, w3_ref[...], preferred_element_type=jnp.float32)
    h = ((h + b3_ref[...]) * _INV) * g3_ref[...] + be3_ref[...]
    x = jnp.maximum(h, 0.0)

    z = jnp.dot(x, w4_ref[...], preferred_element_type=jnp.float32)
    z = z + b4_ref[...]
    out_ref[...] = jax.nn.sigmoid(z)


def _full(shape):
    return pl.BlockSpec(shape, lambda i: (0, 0))


_mlp_call = pl.pallas_call(
    _mlp_body,
    grid=(_B // _BLK,),
    in_specs=[
        pl.BlockSpec((_BLK, _E1), lambda i: (i, 0)),
        pl.BlockSpec((_BLK, _ND), lambda i: (i, 0)),
        _full((_E1, _H1)), _full((_ND, _H1)),
        _full((1, _H1)), _full((1, _H1)), _full((1, _H1)),
        _full((_H1, _H2)), _full((1, _H2)), _full((1, _H2)), _full((1, _H2)),
        _full((_H2, _H3)), _full((1, _H3)), _full((1, _H3)), _full((1, _H3)),
        _full((_H3, 1)), _full((1, 1)),
    ],
    out_specs=pl.BlockSpec((_BLK, 1), lambda i: (i, 0)),
    out_shape=jax.ShapeDtypeStruct((_B, 1), jnp.float32),
)


def kernel(categorical_inputs, numerical_inputs, emb_tables,
           W1, b1, g1, be1, W2, b2, g2, be2, W3, b3, g3, be3, W4, b4):
    cat = jnp.clip(categorical_inputs, 0, _VOCAB - 1).astype(jnp.int32)
    idx = cat + (jnp.arange(_NF, dtype=jnp.int32) * _VOCAB)[None, :]
    idx2d = idx.reshape(_NW * _NCH, _CHUNK)

    tt = jnp.transpose(emb_tables, (0, 2, 1))
    tail = jnp.pad(emb_tables[:, _VT * 128:, :],
                   ((0, 0), (0, 0), (0, _EDP - _ED))).reshape(_NF, 16, 128)
    t64 = _sc_transpose()(tt, tail)
    table_lin = t64.reshape(_NF * _VOCAB, _EDP)

    emb_flat = _sc_gather()(table_lin, idx2d)
    emb = emb_flat.reshape(_B, _E1)

    w1 = W1.T
    w1e = jnp.pad(w1[:_NF * _ED].reshape(_NF, _ED, _H1),
                  ((0, 0), (0, _EDP - _ED), (0, 0))).reshape(_E1, _H1)
    out = _mlp_call(
        emb, numerical_inputs,
        w1e, w1[_NF * _ED:],
        b1.reshape(1, _H1), g1.reshape(1, _H1), be1.reshape(1, _H1),
        W2.T, b2.reshape(1, _H2), g2.reshape(1, _H2), be2.reshape(1, _H2),
        W3.T, b3.reshape(1, _H3), g3.reshape(1, _H3), be3.reshape(1, _H3),
        W4.T, b4.reshape(1, 1),
    )
    return out[:, 0]


# speedup vs baseline: 1.3963x; 1.3963x over previous
import functools
import math

import jax
import jax.numpy as jnp
from jax import lax
from jax.experimental import pallas as pl
from jax.experimental.pallas import tpu as pltpu
from jax.experimental.pallas import tpu_sc as plsc

_B = 16384
_NF = 26
_VOCAB = 100000
_ED = 50
_EDP = 64
_EPS = 1e-5
_INV = 1.0 / math.sqrt(1.0 + _EPS)

_NC, _NS = 2, 16
_NW = _NC * _NS

_VT = _VOCAB // 128
_VREM = _VOCAB - _VT * 128
_NQ = _NF * _VT
_QPW = (_NQ + _NW - 1) // _NW


def _iota16():
    return lax.iota(jnp.int32, 16)


def _t16(xs):
    lane = _iota16()
    for s in (8, 4, 2, 1):
        m = (lane & s) == 0
        up_ix = (lane - s) & 15
        dn_ix = (lane + s) & 15
        ys = list(xs)
        for r in range(16):
            if r & s == 0:
                p, q = xs[r], xs[r + s]
                ys[r] = jnp.where(
                    m, p, q.at[up_ix].get(mode="promise_in_bounds"))
                ys[r + s] = jnp.where(
                    m, p.at[dn_ix].get(mode="promise_in_bounds"), q)
        xs = ys
    return xs


def _tp_block(buf, obuf, nv):
    def vblock(vb2, carry):
        for u in range(2):
            vb = vb2 * 2 + u
            v0 = vb * 16
            for eb in range(4):
                e0 = eb * 16
                xs = [buf[e0 + r, pl.ds(v0, 16)] for r in range(16)]
                ys = _t16(xs)
                for i in range(16):
                    obuf[vb * 8 + i // 2,
                         pl.ds((i % 2) * 64 + e0, 16)] = ys[i]
        return carry

    lax.fori_loop(0, nv // 32, vblock, 0)


_QFULL = _NQ // _NW


def _sc_transpose_body(tt_hbm, tail_hbm, out_hbm,
                       buf_a, buf_b, obuf_a, obuf_b, gsem, ssem):
    wid = lax.axis_index("s") * _NC + lax.axis_index("c")

    def start_in(q, buf):
        f = q // _VT
        c = q % _VT
        d1 = pltpu.async_copy(tt_hbm.at[f, pl.ds(0, 48), pl.ds(c * 128, 128)],
                              buf.at[pl.ds(0, 48)], gsem)
        d2 = pltpu.async_copy(tt_hbm.at[f, pl.ds(48, 2), pl.ds(c * 128, 128)],
                              buf.at[pl.ds(48, 2)], gsem)
        return d1, d2

    def start_out(q, obuf):
        f = q // _VT
        c = q % _VT
        return pltpu.async_copy(
            obuf, out_hbm.at[pl.ds(f * (_VOCAB // 2) + c * 64, 64)], ssem)

    def drain_in(buf):
        pltpu.make_async_copy(
            tt_hbm.at[0, pl.ds(0, 48), pl.ds(0, 128)],
            buf.at[pl.ds(0, 48)], gsem).wait()
        pltpu.make_async_copy(
            tt_hbm.at[0, pl.ds(48, 2), pl.ds(0, 128)],
            buf.at[pl.ds(48, 2)], gsem).wait()

    def drain_out(obuf):
        pltpu.make_async_copy(obuf, out_hbm.at[pl.ds(0, 64)], ssem).wait()

    q_last = wid + _QFULL * _NW

    start_in(wid, buf_a)
    start_in(wid + _NW, buf_b)

    def step(k2, carry):
        qa = wid + (2 * k2) * _NW
        qb = wid + (2 * k2 + 1) * _NW
        drain_in(buf_a)

        @pl.when(k2 > 0)
        def _():
            drain_out(obuf_a)

        _tp_block(buf_a, obuf_a, 128)
        sa = start_out(qa, obuf_a)
        qn_a = qa + 2 * _NW

        @pl.when(qn_a < _NQ)
        def _():
            start_in(qn_a, buf_a)

        drain_in(buf_b)

        @pl.when(k2 > 0)
        def _():
            drain_out(obuf_b)

        _tp_block(buf_b, obuf_b, 128)
        sb = start_out(qb, obuf_b)
        qn_b = qb + 2 * _NW

        @pl.when(qn_b < _NQ)
        def _():
            start_in(qn_b, buf_b)

        return carry

    lax.fori_loop(0, _QFULL // 2, step, 0)
    drain_out(obuf_a)
    drain_out(obuf_b)

    @pl.when(q_last < _NQ)
    def _():
        drain_in(buf_a)
        _tp_block(buf_a, obuf_a, 128)
        start_out(q_last, obuf_a).wait()

    @pl.when(wid < _NF)
    def _():
        f = wid
        pltpu.sync_copy(tail_hbm.at[f], obuf_b.at[pl.ds(0, 16)])
        pltpu.sync_copy(
            obuf_b.at[pl.ds(0, 16)],
            out_hbm.at[pl.ds(f * (_VOCAB // 2) + _VT * 64, 16)])


@functools.cache
def _sc_transpose():
    mesh = plsc.VectorSubcoreMesh(
        core_axis_name="c", subcore_axis_name="s",
        num_cores=_NC, num_subcores=_NS)
    return pl.kernel(
        _sc_transpose_body,
        out_type=jax.ShapeDtypeStruct((_NF * _VOCAB // 2, 128), jnp.float32),
        mesh=mesh,
        scratch_types=[
            pltpu.VMEM((64, 128), jnp.float32),
            pltpu.VMEM((64, 128), jnp.float32),
            pltpu.VMEM((64, 128), jnp.float32),
            pltpu.VMEM((64, 128), jnp.float32),
            pltpu.SemaphoreType.DMA,
            pltpu.SemaphoreType.DMA,
        ],
        compiler_params=pltpu.CompilerParams(
            use_tc_tiling_on_sc=True, needs_layout_passes=False),
    )


_ROWS = _B * _NF
_RPW = _ROWS // _NW
_CHUNK = 128
_NCH = _RPW // _CHUNK
_K = 8


def _sc_gather_body(table_hbm, idx_hbm, out_hbm, idx_v, rows_v, gsem, ssem):
    wid = lax.axis_index("s") * _NC + lax.axis_index("c")
    pltpu.sync_copy(idx_hbm.at[pl.ds(wid * _NCH, _NCH)], idx_v)

    def group(g, carry):
        c0 = g * _K
        gd = []
        for b in range(_K):
            gd.append(pltpu.async_copy(
                table_hbm.at[idx_v.at[c0 + b]], rows_v.at[b], gsem))
        sd = []
        for b in range(_K):
            gd[b].wait()
            sd.append(pltpu.async_copy(
                rows_v.at[b],
                out_hbm.at[pl.ds((wid * _NCH + c0 + b) * _CHUNK, _CHUNK)],
                ssem))
        for b in range(_K):
            sd[b].wait()
        return carry

    lax.fori_loop(0, _NCH // _K, group, 0)


@functools.cache
def _sc_gather():
    mesh = plsc.VectorSubcoreMesh(
        core_axis_name="c", subcore_axis_name="s",
        num_cores=_NC, num_subcores=_NS)
    return pl.kernel(
        _sc_gather_body,
        out_type=jax.ShapeDtypeStruct((_ROWS, _EDP), jnp.float32),
        mesh=mesh,
        scratch_types=[
            pltpu.VMEM((_NCH, _CHUNK), jnp.int32),
            pltpu.VMEM((_K, _CHUNK, _EDP), jnp.float32),
            pltpu.SemaphoreType.DMA,
            pltpu.SemaphoreType.DMA,
        ],
        compiler_params=pltpu.CompilerParams(use_tc_tiling_on_sc=False),
    )


_BLK = 512
_E1 = _NF * _EDP
_ND = 13
_H1, _H2, _H3 = 512, 256, 128


def _mlp_body(emb_ref, num_ref, w1e_ref, w1n_ref, b1_ref, g1_ref, be1_ref,
              w2_ref, b2_ref, g2_ref, be2_ref,
              w3_ref, b3_ref, g3_ref, be3_ref,
              w4_ref, b4_ref, out_ref):
    h = jnp.dot(emb_ref[...], w1e_ref[...], preferred_element_type=jnp.float32)
    h = h + jnp.dot(num_ref[...], w1n_ref[...],
                    preferred_element_type=jnp.float32)
    h = ((h + b1_ref[...]) * _INV) * g1_ref[...] + be1_ref[...]
    x = jnp.maximum(h, 0.0)

    h = jnp.dot(x, w2_ref[...], preferred_element_type=jnp.float32)
    h = ((h + b2_ref[...]) * _INV) * g2_ref[...] + be2_ref[...]
    x = jnp.maximum(h, 0.0)

    h = jnp.dot(x, w3_ref[...], preferred_element_type=jnp.float32)
    h = ((h + b3_ref[...]) * _INV) * g3_ref[...] + be3_ref[...]
    x = jnp.maximum(h, 0.0)

    z = jnp.dot(x, w4_ref[...], preferred_element_type=jnp.float32)
    z = z + b4_ref[...]
    out_ref[...] = jax.nn.sigmoid(z)


def _full(shape):
    return pl.BlockSpec(shape, lambda i: (0, 0))


_mlp_call = pl.pallas_call(
    _mlp_body,
    grid=(_B // _BLK,),
    in_specs=[
        pl.BlockSpec((_BLK, _E1), lambda i: (i, 0)),
        pl.BlockSpec((_BLK, _ND), lambda i: (i, 0)),
        _full((_E1, _H1)), _full((_ND, _H1)),
        _full((1, _H1)), _full((1, _H1)), _full((1, _H1)),
        _full((_H1, _H2)), _full((1, _H2)), _full((1, _H2)), _full((1, _H2)),
        _full((_H2, _H3)), _full((1, _H3)), _full((1, _H3)), _full((1, _H3)),
        _full((_H3, 1)), _full((1, 1)),
    ],
    out_specs=pl.BlockSpec((_BLK, 1), lambda i: (i, 0)),
    out_shape=jax.ShapeDtypeStruct((_B, 1), jnp.float32),
)


def kernel(categorical_inputs, numerical_inputs, emb_tables,
           W1, b1, g1, be1, W2, b2, g2, be2, W3, b3, g3, be3, W4, b4):
    cat = jnp.clip(categorical_inputs, 0, _VOCAB - 1).astype(jnp.int32)
    idx = cat + (jnp.arange(_NF, dtype=jnp.int32) * _VOCAB)[None, :]
    idx2d = idx.reshape(_NW * _NCH, _CHUNK)

    tt = jnp.transpose(emb_tables, (0, 2, 1))
    tail = jnp.pad(emb_tables[:, _VT * 128:, :],
                   ((0, 0), (0, 0), (0, _EDP - _ED))).reshape(_NF, 16, 128)
    t64 = _sc_transpose()(tt, tail)
    table_lin = t64.reshape(_NF * _VOCAB, _EDP)

    emb_flat = _sc_gather()(table_lin, idx2d)
    emb = emb_flat.reshape(_B, _E1)

    w1 = W1.T
    w1e = jnp.pad(w1[:_NF * _ED].reshape(_NF, _ED, _H1),
                  ((0, 0), (0, _EDP - _ED), (0, 0))).reshape(_E1, _H1)
    out = _mlp_call(
        emb, numerical_inputs,
        w1e, w1[_NF * _ED:],
        b1.reshape(1, _H1), g1.reshape(1, _H1), be1.reshape(1, _H1),
        W2.T, b2.reshape(1, _H2), g2.reshape(1, _H2), be2.reshape(1, _H2),
        W3.T, b3.reshape(1, _H3), g3.reshape(1, _H3), be3.reshape(1, _H3),
        W4.T, b4.reshape(1, 1),
    )
    return out[:, 0]


# speedup vs baseline: 1.3970x; 1.0005x over previous
import functools
import math

import jax
import jax.numpy as jnp
from jax import lax
from jax.experimental import pallas as pl
from jax.experimental.pallas import tpu as pltpu
from jax.experimental.pallas import tpu_sc as plsc

_B = 16384
_NF = 26
_VOCAB = 100000
_ED = 50
_EDP = 64
_EPS = 1e-5
_INV = 1.0 / math.sqrt(1.0 + _EPS)

_NC, _NS = 2, 16
_NW = _NC * _NS

_VT = _VOCAB // 128
_VREM = _VOCAB - _VT * 128
_NQ = _NF * _VT


def _iota16():
    return lax.iota(jnp.int32, 16)


def _t16(xs):
    lane = _iota16()
    for s in (8, 4, 2, 1):
        m = (lane & s) == 0
        up_ix = (lane - s) & 15
        dn_ix = (lane + s) & 15
        ys = list(xs)
        for r in range(16):
            if r & s == 0:
                p, q = xs[r], xs[r + s]
                ys[r] = jnp.where(
                    m, p, q.at[up_ix].get(mode="promise_in_bounds"))
                ys[r + s] = jnp.where(
                    m, p.at[dn_ix].get(mode="promise_in_bounds"), q)
        xs = ys
    return xs


def _tp_block(buf, obuf, nv):
    def vblock(vb2, carry):
        for u in range(2):
            vb = vb2 * 2 + u
            v0 = vb * 16
            for eb in range(4):
                e0 = eb * 16
                xs = [buf[e0 + r, pl.ds(v0, 16)] for r in range(16)]
                ys = _t16(xs)
                for i in range(16):
                    obuf[vb * 8 + i // 2,
                         pl.ds((i % 2) * 64 + e0, 16)] = ys[i]
        return carry

    lax.fori_loop(0, nv // 32, vblock, 0)


_QFULL = _NQ // _NW


def _sc_transpose_body(tt_hbm, tail_hbm, out_hbm,
                       buf_a, buf_b, obuf_a, obuf_b, gsem, ssem):
    wid = lax.axis_index("s") * _NC + lax.axis_index("c")

    def start_in(q, buf):
        f = q // _VT
        c = q % _VT
        d1 = pltpu.async_copy(tt_hbm.at[f, pl.ds(0, 48), pl.ds(c * 128, 128)],
                              buf.at[pl.ds(0, 48)], gsem)
        d2 = pltpu.async_copy(tt_hbm.at[f, pl.ds(48, 2), pl.ds(c * 128, 128)],
                              buf.at[pl.ds(48, 2)], gsem)
        return d1, d2

    def start_out(q, obuf):
        f = q // _VT
        c = q % _VT
        return pltpu.async_copy(
            obuf, out_hbm.at[pl.ds(f * (_VOCAB // 2) + c * 64, 64)], ssem)

    def drain_in(buf):
        pltpu.make_async_copy(
            tt_hbm.at[0, pl.ds(0, 48), pl.ds(0, 128)],
            buf.at[pl.ds(0, 48)], gsem).wait()
        pltpu.make_async_copy(
            tt_hbm.at[0, pl.ds(48, 2), pl.ds(0, 128)],
            buf.at[pl.ds(48, 2)], gsem).wait()

    def drain_out(obuf):
        pltpu.make_async_copy(obuf, out_hbm.at[pl.ds(0, 64)], ssem).wait()

    q_last = wid + _QFULL * _NW

    start_in(wid, buf_a)
    start_in(wid + _NW, buf_b)

    def step(k2, carry):
        qa = wid + (2 * k2) * _NW
        qb = wid + (2 * k2 + 1) * _NW
        drain_in(buf_a)

        @pl.when(k2 > 0)
        def _():
            drain_out(obuf_a)

        _tp_block(buf_a, obuf_a, 128)
        sa = start_out(qa, obuf_a)
        qn_a = qa + 2 * _NW

        @pl.when(qn_a < _NQ)
        def _():
            start_in(qn_a, buf_a)

        drain_in(buf_b)

        @pl.when(k2 > 0)
        def _():
            drain_out(obuf_b)

        _tp_block(buf_b, obuf_b, 128)
        sb = start_out(qb, obuf_b)
        qn_b = qb + 2 * _NW

        @pl.when(qn_b < _NQ)
        def _():
            start_in(qn_b, buf_b)

        return carry

    lax.fori_loop(0, _QFULL // 2, step, 0)
    drain_out(obuf_a)
    drain_out(obuf_b)

    @pl.when(q_last < _NQ)
    def _():
        drain_in(buf_a)
        _tp_block(buf_a, obuf_a, 128)
        start_out(q_last, obuf_a).wait()

    @pl.when(wid < _NF)
    def _():
        f = wid
        pltpu.sync_copy(tail_hbm.at[f], obuf_b.at[pl.ds(0, 16)])
        pltpu.sync_copy(
            obuf_b.at[pl.ds(0, 16)],
            out_hbm.at[pl.ds(f * (_VOCAB // 2) + _VT * 64, 16)])


@functools.cache
def _sc_transpose():
    mesh = plsc.VectorSubcoreMesh(
        core_axis_name="c", subcore_axis_name="s",
        num_cores=_NC, num_subcores=_NS)
    return pl.kernel(
        _sc_transpose_body,
        out_type=jax.ShapeDtypeStruct((_NF * _VOCAB // 2, 128), jnp.float32),
        mesh=mesh,
        scratch_types=[
            pltpu.VMEM((64, 128), jnp.float32),
            pltpu.VMEM((64, 128), jnp.float32),
            pltpu.VMEM((64, 128), jnp.float32),
            pltpu.VMEM((64, 128), jnp.float32),
            pltpu.SemaphoreType.DMA,
            pltpu.SemaphoreType.DMA,
        ],
        compiler_params=pltpu.CompilerParams(
            use_tc_tiling_on_sc=True, needs_layout_passes=False),
    )


_ROWS = _B * _NF
_RPW = _ROWS // _NW
_CHUNK = 128
_NCH = _RPW // _CHUNK
_K = 8


def _sc_gather_body(table_hbm, idx_hbm, out_hbm, idx_v, rows_v, gsem, ssem):
    wid = lax.axis_index("s") * _NC + lax.axis_index("c")
    pltpu.sync_copy(idx_hbm.at[pl.ds(wid * _NCH, _NCH)], idx_v)

    def group(g, carry):
        c0 = g * _K
        gd = []
        for b in range(_K):
            gd.append(pltpu.async_copy(
                table_hbm.at[idx_v.at[c0 + b]], rows_v.at[b], gsem))
        sd = []
        for b in range(_K):
            gd[b].wait()
            sd.append(pltpu.async_copy(
                rows_v.at[b],
                out_hbm.at[pl.ds((wid * _NCH + c0 + b) * _CHUNK, _CHUNK)],
                ssem))
        for b in range(_K):
            sd[b].wait()
        return carry

    lax.fori_loop(0, _NCH // _K, group, 0)


@functools.cache
def _sc_gather():
    mesh = plsc.VectorSubcoreMesh(
        core_axis_name="c", subcore_axis_name="s",
        num_cores=_NC, num_subcores=_NS)
    return pl.kernel(
        _sc_gather_body,
        out_type=jax.ShapeDtypeStruct((_ROWS, _EDP), jnp.float32),
        mesh=mesh,
        scratch_types=[
            pltpu.VMEM((_NCH, _CHUNK), jnp.int32),
            pltpu.VMEM((_K, _CHUNK, _EDP), jnp.float32),
            pltpu.SemaphoreType.DMA,
            pltpu.SemaphoreType.DMA,
        ],
        compiler_params=pltpu.CompilerParams(use_tc_tiling_on_sc=False),
    )


_BLK = 512
_E1 = _NF * _EDP
_ND = 13
_H1, _H2, _H3 = 512, 256, 128


def _mlp_body(emb_ref, num_ref, w1e_ref, w1n_ref, b1_ref, g1_ref, be1_ref,
              w2_ref, b2_ref, g2_ref, be2_ref,
              w3_ref, b3_ref, g3_ref, be3_ref,
              w4_ref, b4_ref, out_ref):
    h = jnp.dot(emb_ref[...], w1e_ref[...], preferred_element_type=jnp.float32)
    h = h + jnp.dot(num_ref[...], w1n_ref[...],
                    preferred_element_type=jnp.float32)
    h = ((h + b1_ref[...]) * _INV) * g1_ref[...] + be1_ref[...]
    x = jnp.maximum(h, 0.0)

    h = jnp.dot(x, w2_ref[...], preferred_element_type=jnp.float32)
    h = ((h + b2_ref[...]) * _INV) * g2_ref[...] + be2_ref[...]
    x = jnp.maximum(h, 0.0)

    h = jnp.dot(x, w3_ref[...], preferred_element_type=jnp.float32)
    h = ((h + b3_ref[...]) * _INV) * g3_ref[...] + be3_ref[...]
    x = jnp.maximum(h, 0.0)

    z = jnp.dot(x, w4_ref[...], preferred_element_type=jnp.float32)
    z = z + b4_ref[...]
    out_ref[...] = jax.nn.sigmoid(z)


def _full(shape):
    return pl.BlockSpec(shape, lambda i: (0, 0))


_mlp_call = pl.pallas_call(
    _mlp_body,
    grid=(_B // _BLK,),
    in_specs=[
        pl.BlockSpec((_BLK, _E1), lambda i: (i, 0)),
        pl.BlockSpec((_BLK, _ND), lambda i: (i, 0)),
        _full((_E1, _H1)), _full((_ND, _H1)),
        _full((1, _H1)), _full((1, _H1)), _full((1, _H1)),
        _full((_H1, _H2)), _full((1, _H2)), _full((1, _H2)), _full((1, _H2)),
        _full((_H2, _H3)), _full((1, _H3)), _full((1, _H3)), _full((1, _H3)),
        _full((_H3, 1)), _full((1, 1)),
    ],
    out_specs=pl.BlockSpec((_BLK, 1), lambda i: (i, 0)),
    out_shape=jax.ShapeDtypeStruct((_B, 1), jnp.float32),
)


def kernel(categorical_inputs, numerical_inputs, emb_tables,
           W1, b1, g1, be1, W2, b2, g2, be2, W3, b3, g3, be3, W4, b4):
    cat = jnp.clip(categorical_inputs, 0, _VOCAB - 1).astype(jnp.int32)
    idx = cat + (jnp.arange(_NF, dtype=jnp.int32) * _VOCAB)[None, :]
    idx2d = idx.reshape(_NW * _NCH, _CHUNK)

    tt = jnp.transpose(emb_tables, (0, 2, 1))
    tail = jnp.pad(emb_tables[:, _VT * 128:, :],
                   ((0, 0), (0, 0), (0, _EDP - _ED))).reshape(_NF, 16, 128)
    t64 = _sc_transpose()(tt, tail)
    table_lin = t64.reshape(_NF * _VOCAB, _EDP)

    emb_flat = _sc_gather()(table_lin, idx2d)
    emb = emb_flat.reshape(_B, _E1)

    w1 = W1.T
    w1e = jnp.pad(w1[:_NF * _ED].reshape(_NF, _ED, _H1),
                  ((0, 0), (0, _EDP - _ED), (0, 0))).reshape(_E1, _H1)
    out = _mlp_call(
        emb, numerical_inputs,
        w1e, w1[_NF * _ED:],
        b1.reshape(1, _H1), g1.reshape(1, _H1), be1.reshape(1, _H1),
        W2.T, b2.reshape(1, _H2), g2.reshape(1, _H2), be2.reshape(1, _H2),
        W3.T, b3.reshape(1, _H3), g3.reshape(1, _H3), be3.reshape(1, _H3),
        W4.T, b4.reshape(1, 1),
    )
    return out[:, 0]
